# baseline (speedup 1.0000x reference)
import functools
import jax
import jax.numpy as jnp
from jax import lax
from jax.experimental import pallas as pl
from jax.experimental.pallas import tpu as pltpu
from jax.experimental.pallas import tpu_sc as plsc

_D = 128
_N = 10000
_E = 320000
_EPS = 1e-5
_EBLK = 640
_NBLK = 1000

_NW = 32
_GCHUNK = 400


def _sc_gather_sorted(table, ve, src, sdst, perm):
    per_w = _E // _NW
    n_it = per_w // _GCHUNK
    mesh = plsc.VectorSubcoreMesh(core_axis_name="c", subcore_axis_name="s")

    @functools.partial(
        pl.kernel,
        out_type=[jax.ShapeDtypeStruct((_E, _D), jnp.float32),
                  jax.ShapeDtypeStruct((_E, _D), jnp.float32),
                  jax.ShapeDtypeStruct((_E, _D), jnp.float32)],
        mesh=mesh,
        scratch_types=[
            pltpu.VMEM((_GCHUNK,), jnp.int32),
            pltpu.VMEM((_GCHUNK,), jnp.int32),
            pltpu.VMEM((_GCHUNK,), jnp.int32),
            pltpu.VMEM((_GCHUNK, _D), jnp.float32),
            pltpu.SemaphoreType.DMA,
        ],
    )
    def k(table_hbm, ve_hbm, src_hbm, sdst_hbm, perm_hbm,
          a_hbm, b_hbm, veg_hbm, perm_v, sdst_v, srcg_v, rows_v, sem):
        wid = lax.axis_index("s") * 2 + lax.axis_index("c")
        base0 = wid * per_w

        def body(j, carry):
            base = base0 + j * _GCHUNK
            sl = pl.ds(base, _GCHUNK)
            pltpu.sync_copy(perm_hbm.at[sl], perm_v)
            pltpu.sync_copy(sdst_hbm.at[sl], sdst_v)
            pltpu.async_copy(src_hbm.at[perm_v], srcg_v, sem).wait()
            pltpu.async_copy(table_hbm.at[srcg_v], rows_v, sem).wait()
            pltpu.sync_copy(rows_v, a_hbm.at[sl])
            pltpu.async_copy(table_hbm.at[sdst_v], rows_v, sem).wait()
            pltpu.sync_copy(rows_v, b_hbm.at[sl])
            pltpu.async_copy(ve_hbm.at[perm_v], rows_v, sem).wait()
            pltpu.sync_copy(rows_v, veg_hbm.at[sl])
            return carry

        lax.fori_loop(0, n_it, body, 0)

    return k(table, ve, src, sdst, perm)


def _sc_unpermute(x, perm):
    per_w = _E // _NW
    n_it = per_w // _GCHUNK
    mesh = plsc.VectorSubcoreMesh(core_axis_name="c", subcore_axis_name="s")

    @functools.partial(
        pl.kernel,
        out_type=jax.ShapeDtypeStruct((_E, _D), jnp.float32),
        mesh=mesh,
        scratch_types=[
            pltpu.VMEM((_GCHUNK,), jnp.int32),
            pltpu.VMEM((_GCHUNK, _D), jnp.float32),
            pltpu.SemaphoreType.DMA,
        ],
    )
    def k(x_hbm, perm_hbm, out_hbm, perm_v, rows_v, sem):
        wid = lax.axis_index("s") * 2 + lax.axis_index("c")
        base0 = wid * per_w

        def body(j, carry):
            base = base0 + j * _GCHUNK
            sl = pl.ds(base, _GCHUNK)
            pltpu.sync_copy(perm_hbm.at[sl], perm_v)
            pltpu.sync_copy(x_hbm.at[sl], rows_v)
            pltpu.async_copy(rows_v, out_hbm.at[perm_v], sem).wait()
            return carry

        lax.fori_loop(0, n_it, body, 0)

    return k(x, perm)


def _inclusive_cumsum(x):
    n = x.shape[0]
    s = x
    shift = 1
    while shift < n:
        z = jnp.zeros((shift, x.shape[1]), x.dtype)
        s = s + jnp.concatenate([z, s[:-shift]], axis=0)
        shift *= 2
    return s


def _cumsum_kernel(x1_ref, x4_ref, c1_ref, c4_ref, carry_ref):
    @pl.when(pl.program_id(0) == 0)
    def _():
        carry_ref[...] = jnp.zeros_like(carry_ref)

    cs1 = _inclusive_cumsum(x1_ref[...]) + carry_ref[0:1, :]
    c1_ref[...] = cs1
    carry_ref[0:1, :] = cs1[-1:, :]
    cs4 = _inclusive_cumsum(x4_ref[...]) + carry_ref[1:2, :]
    c4_ref[...] = cs4
    carry_ref[1:2, :] = cs4[-1:, :]


def _cumsum2(x1, x4):
    n = x1.shape[0]
    return pl.pallas_call(
        _cumsum_kernel,
        grid=(n // _EBLK,),
        in_specs=[
            pl.BlockSpec((_EBLK, _D), lambda i: (i, 0)),
            pl.BlockSpec((_EBLK, _D), lambda i: (i, 0)),
        ],
        out_specs=[
            pl.BlockSpec((_EBLK, _D), lambda i: (i, 0)),
            pl.BlockSpec((_EBLK, _D), lambda i: (i, 0)),
        ],
        out_shape=[
            jax.ShapeDtypeStruct((n, _D), jnp.float32),
            jax.ShapeDtypeStruct((n, _D), jnp.float32),
        ],
        scratch_shapes=[pltpu.VMEM((8, _D), jnp.float32)],
    )(x1, x4)


def _edge_kernel(a_ref, b_ref, ve_ref, W1_ref, b1_ref, W2p_ref, b2p_ref,
                 We1_ref, We2_ref, be_ref, f1_ref, f2_ref, f3_ref, f4_ref,
                 vepre_ref, stats_ref):
    x = jnp.concatenate([a_ref[...], b_ref[...], ve_ref[...]], axis=1)
    h = jnp.maximum(x @ W1_ref[...] + b1_ref[...], 0.0)
    m = h @ W2p_ref[...] + b2p_ref[...]
    k = jax.nn.sigmoid(m[:, 4 * _D:4 * _D + 1])
    f = m[:, :4 * _D] * k
    f1_ref[...] = f[:, :_D]
    f2_ref[...] = f[:, _D:2 * _D]
    f3_ref[...] = f[:, 2 * _D:3 * _D]
    f4_ref[...] = f[:, 3 * _D:]
    vp = f @ We1_ref[...] + ve_ref[...] @ We2_ref[...] + be_ref[...]
    vepre_ref[...] = vp

    @pl.when(pl.program_id(0) == 0)
    def _():
        stats_ref[...] = jnp.zeros_like(stats_ref)

    s = jnp.sum(vp, axis=0, keepdims=True)
    s2 = jnp.sum(vp * vp, axis=0, keepdims=True)
    stats_ref[0:1, :] += s
    stats_ref[1:2, :] += s2


def _node_kernel(vc_ref, nf1_ref, nf2_ref, nf3_ref, nf4_ref, Wr_ref, br_ref,
                 out_ref, stats_ref):
    Wr = Wr_ref[...]
    o = (vc_ref[...] @ Wr[0:_D] + nf1_ref[...] @ Wr[_D:2 * _D]
         + nf2_ref[...] @ Wr[2 * _D:3 * _D] + nf3_ref[...] @ Wr[3 * _D:4 * _D]
         + nf4_ref[...] @ Wr[4 * _D:5 * _D] + br_ref[...])
    out_ref[...] = o

    @pl.when(pl.program_id(0) == 0)
    def _():
        stats_ref[...] = jnp.zeros_like(stats_ref)

    stats_ref[0:1, :] += jnp.sum(o, axis=0, keepdims=True)
    stats_ref[1:2, :] += jnp.sum(o * o, axis=0, keepdims=True)


def _norm_kernel(x_ref, scale_ref, shift_ref, o_ref):
    o_ref[...] = x_ref[...] * scale_ref[...] + shift_ref[...]


def _normalize(x, scale, shift, blk):
    n = x.shape[0]
    return pl.pallas_call(
        _norm_kernel,
        grid=(n // blk,),
        in_specs=[
            pl.BlockSpec((blk, _D), lambda i: (i, 0)),
            pl.BlockSpec((1, _D), lambda i: (0, 0)),
            pl.BlockSpec((1, _D), lambda i: (0, 0)),
        ],
        out_specs=pl.BlockSpec((blk, _D), lambda i: (i, 0)),
        out_shape=jax.ShapeDtypeStruct((n, _D), jnp.float32),
    )(x, scale, shift)


def kernel(in_vc, in_ve, edge_index, W1, b1, W2, b2, Wr, br, We, be,
           gamma_gc, beta_gc, gamma_ef, beta_ef):
    src = edge_index[0]
    dst = edge_index[1]

    W2p = jnp.concatenate(
        [W2[:, 1:], W2[:, 0:1], jnp.zeros((3 * _D, _D - 1), jnp.float32)], axis=1)
    b2p = jnp.concatenate(
        [b2[1:], b2[0:1], jnp.zeros((_D - 1,), jnp.float32)])[None, :]
    b1r = b1[None, :]
    ber = be[None, :]
    brr = br[None, :]
    We1 = We[:4 * _D]
    We2 = We[4 * _D:]

    sdst, perm = lax.sort_key_val(dst, lax.iota(jnp.int32, _E))

    a, b, veg = _sc_gather_sorted(in_vc, in_ve, src, sdst, perm)

    deg = jax.ops.segment_sum(jnp.ones((_E,), jnp.float32), dst,
                              num_segments=_N)

    half = _E // 2
    nhb = half // _EBLK

    def _edge_half(off):
        return pl.pallas_call(
            _edge_kernel,
            grid=(nhb,),
            in_specs=[
                pl.BlockSpec((_EBLK, _D), lambda i: (i + off, 0)),
                pl.BlockSpec((_EBLK, _D), lambda i: (i + off, 0)),
                pl.BlockSpec((_EBLK, _D), lambda i: (i + off, 0)),
                pl.BlockSpec((3 * _D, 3 * _D), lambda i: (0, 0)),
                pl.BlockSpec((1, 3 * _D), lambda i: (0, 0)),
                pl.BlockSpec((3 * _D, 5 * _D), lambda i: (0, 0)),
                pl.BlockSpec((1, 5 * _D), lambda i: (0, 0)),
                pl.BlockSpec((4 * _D, _D), lambda i: (0, 0)),
                pl.BlockSpec((_D, _D), lambda i: (0, 0)),
                pl.BlockSpec((1, _D), lambda i: (0, 0)),
            ],
            out_specs=[
                pl.BlockSpec((_EBLK, _D), lambda i: (i, 0)),
                pl.BlockSpec((_EBLK, _D), lambda i: (i, 0)),
                pl.BlockSpec((_EBLK, _D), lambda i: (i, 0)),
                pl.BlockSpec((_EBLK, _D), lambda i: (i, 0)),
                pl.BlockSpec((_EBLK, _D), lambda i: (i, 0)),
                pl.BlockSpec((8, _D), lambda i: (0, 0)),
            ],
            out_shape=[
                jax.ShapeDtypeStruct((half, _D), jnp.float32),
                jax.ShapeDtypeStruct((half, _D), jnp.float32),
                jax.ShapeDtypeStruct((half, _D), jnp.float32),
                jax.ShapeDtypeStruct((half, _D), jnp.float32),
                jax.ShapeDtypeStruct((half, _D), jnp.float32),
                jax.ShapeDtypeStruct((8, _D), jnp.float32),
            ],
        )(a, b, veg, W1, b1r, W2p, b2p, We1, We2, ber)

    sdst1 = sdst[:half]
    sdst2 = sdst[half:]
    f1a, f2a, f3a, f4a, vpa, esta = _edge_half(0)
    C1a, C4a = _cumsum2(f1a, f4a)
    m2a = jax.ops.segment_max(f2a, sdst1, num_segments=_N,
                              indices_are_sorted=True)
    m3a = jax.ops.segment_min(f3a, sdst1, num_segments=_N,
                              indices_are_sorted=True)
    f1b, f2b, f3b, f4b, vpb, estb = _edge_half(nhb)
    C1b, C4b = _cumsum2(f1b, f4b)
    m2b = jax.ops.segment_max(f2b, sdst2, num_segments=_N,
                              indices_are_sorted=True)
    m3b = jax.ops.segment_min(f3b, sdst2, num_segments=_N,
                              indices_are_sorted=True)

    rp_end = jnp.cumsum(deg).astype(jnp.int32)
    rp_start = rp_end - deg.astype(jnp.int32)

    T1a = C1a[half - 1]
    T4a = C4a[half - 1]

    def _cum_at(Ca, Cb, Ta, idx):
        va = jnp.take(Ca, jnp.clip(idx - 1, 0, half - 1), axis=0,
                      mode="clip")
        vb = jnp.take(Cb, jnp.clip(idx - half - 1, 0, half - 1), axis=0,
                      mode="clip") + Ta[None, :]
        v = jnp.where((idx > half)[:, None], vb, va)
        return jnp.where((idx == 0)[:, None], 0.0, v)

    nf1 = (_cum_at(C1a, C1b, T1a, rp_end)
           - _cum_at(C1a, C1b, T1a, rp_start))
    s4 = (_cum_at(C4a, C4b, T4a, rp_end)
          - _cum_at(C4a, C4b, T4a, rp_start))

    vepre = jnp.concatenate([vpa, vpb], axis=0)
    est = esta + estb

    has = (deg > 0)[:, None]
    nf2 = jnp.where(has, jnp.maximum(m2a, m2b), 0.0)
    nf3 = jnp.where(has, jnp.minimum(m3a, m3b), 0.0)
    nf4 = s4 / jnp.maximum(deg, 1.0)[:, None]

    vcpre, nst = pl.pallas_call(
        _node_kernel,
        grid=(_N // _NBLK,),
        in_specs=[
            pl.BlockSpec((_NBLK, _D), lambda i: (i, 0)),
            pl.BlockSpec((_NBLK, _D), lambda i: (i, 0)),
            pl.BlockSpec((_NBLK, _D), lambda i: (i, 0)),
            pl.BlockSpec((_NBLK, _D), lambda i: (i, 0)),
            pl.BlockSpec((_NBLK, _D), lambda i: (i, 0)),
            pl.BlockSpec((5 * _D, _D), lambda i: (0, 0)),
            pl.BlockSpec((1, _D), lambda i: (0, 0)),
        ],
        out_specs=[
            pl.BlockSpec((_NBLK, _D), lambda i: (i, 0)),
            pl.BlockSpec((8, _D), lambda i: (0, 0)),
        ],
        out_shape=[
            jax.ShapeDtypeStruct((_N, _D), jnp.float32),
            jax.ShapeDtypeStruct((8, _D), jnp.float32),
        ],
    )(in_vc, nf1, nf2, nf3, nf4, Wr, brr)

    def _bn_coeffs(stats, n, gamma, beta):
        mu = stats[0] / n
        var = stats[1] / n - mu * mu
        rstd = jax.lax.rsqrt(var + _EPS)
        scale = gamma * rstd
        shift = beta - mu * scale
        return scale[None, :], shift[None, :]

    esc, esh = _bn_coeffs(est, float(_E), gamma_ef, beta_ef)
    nsc, nsh = _bn_coeffs(nst, float(_N), gamma_gc, beta_gc)

    out_vc = _normalize(vcpre, nsc, nsh, _NBLK)
    out_ve = _sc_unpermute(_normalize(vepre, esc, esh, 2000), perm)
    return (out_vc, out_ve)


# speedup vs baseline: 1.4405x; 1.4405x over previous
import functools
import jax
import jax.numpy as jnp
from jax import lax
from jax.experimental import pallas as pl
from jax.experimental.pallas import tpu as pltpu
from jax.experimental.pallas import tpu_sc as plsc

_D = 128
_N = 10000
_E = 320000
_EPS = 1e-5
_EBLK = 640
_NBLK = 1000

_NW = 32
_GCHUNK = 400


def _sc_gather_sorted(table, ve, src, sdst, perm):
    per_w = _E // _NW
    n_it = per_w // _GCHUNK
    mesh = plsc.VectorSubcoreMesh(core_axis_name="c", subcore_axis_name="s")

    @functools.partial(
        pl.kernel,
        out_type=[jax.ShapeDtypeStruct((_E, _D), jnp.float32),
                  jax.ShapeDtypeStruct((_E, _D), jnp.float32),
                  jax.ShapeDtypeStruct((_E, _D), jnp.float32)],
        mesh=mesh,
        scratch_types=[
            pltpu.VMEM((_GCHUNK,), jnp.int32),
            pltpu.VMEM((_GCHUNK,), jnp.int32),
            pltpu.VMEM((_GCHUNK,), jnp.int32),
            pltpu.VMEM((_GCHUNK, _D), jnp.float32),
            pltpu.SemaphoreType.DMA,
        ],
    )
    def k(table_hbm, ve_hbm, src_hbm, sdst_hbm, perm_hbm,
          a_hbm, b_hbm, veg_hbm, perm_v, sdst_v, srcg_v, rows_v, sem):
        wid = lax.axis_index("s") * 2 + lax.axis_index("c")
        base0 = wid * per_w

        def body(j, carry):
            base = base0 + j * _GCHUNK
            sl = pl.ds(base, _GCHUNK)
            pltpu.sync_copy(perm_hbm.at[sl], perm_v)
            pltpu.sync_copy(sdst_hbm.at[sl], sdst_v)
            pltpu.async_copy(src_hbm.at[perm_v], srcg_v, sem).wait()
            pltpu.async_copy(table_hbm.at[srcg_v], rows_v, sem).wait()
            pltpu.sync_copy(rows_v, a_hbm.at[sl])
            pltpu.async_copy(table_hbm.at[sdst_v], rows_v, sem).wait()
            pltpu.sync_copy(rows_v, b_hbm.at[sl])
            pltpu.async_copy(ve_hbm.at[perm_v], rows_v, sem).wait()
            pltpu.sync_copy(rows_v, veg_hbm.at[sl])
            return carry

        lax.fori_loop(0, n_it, body, 0)

    return k(table, ve, src, sdst, perm)


def _sc_unpermute(x, perm):
    per_w = _E // _NW
    n_it = per_w // _GCHUNK
    mesh = plsc.VectorSubcoreMesh(core_axis_name="c", subcore_axis_name="s")

    @functools.partial(
        pl.kernel,
        out_type=jax.ShapeDtypeStruct((_E, _D), jnp.float32),
        mesh=mesh,
        scratch_types=[
            pltpu.VMEM((_GCHUNK,), jnp.int32),
            pltpu.VMEM((_GCHUNK, _D), jnp.float32),
            pltpu.SemaphoreType.DMA,
        ],
    )
    def k(x_hbm, perm_hbm, out_hbm, perm_v, rows_v, sem):
        wid = lax.axis_index("s") * 2 + lax.axis_index("c")
        base0 = wid * per_w

        def body(j, carry):
            base = base0 + j * _GCHUNK
            sl = pl.ds(base, _GCHUNK)
            pltpu.sync_copy(perm_hbm.at[sl], perm_v)
            pltpu.sync_copy(x_hbm.at[sl], rows_v)
            pltpu.async_copy(rows_v, out_hbm.at[perm_v], sem).wait()
            return carry

        lax.fori_loop(0, n_it, body, 0)

    return k(x, perm)


def _inclusive_cumsum(x):
    n = x.shape[0]
    s = x
    shift = 1
    while shift < n:
        z = jnp.zeros((shift, x.shape[1]), x.dtype)
        s = s + jnp.concatenate([z, s[:-shift]], axis=0)
        shift *= 2
    return s


_NPAD = 10240


def _sc_boundary_gather(C1a, C1b, C4a, C4b, ia_s, ib_s, ia_e, ib_e):
    per_w = _NPAD // _NW
    mesh = plsc.VectorSubcoreMesh(core_axis_name="c", subcore_axis_name="s")
    ot = jax.ShapeDtypeStruct((_NPAD, _D), jnp.float32)

    @functools.partial(
        pl.kernel,
        out_type=[ot] * 8,
        mesh=mesh,
        scratch_types=[
            pltpu.VMEM((per_w,), jnp.int32),
            pltpu.VMEM((per_w, _D), jnp.float32),
            pltpu.SemaphoreType.DMA,
        ],
    )
    def k(c1a, c1b, c4a, c4b, sa, sb, ea, eb,
          o1sa, o1sb, o1ea, o1eb, o4sa, o4sb, o4ea, o4eb, idx_v, rows_v, sem):
        wid = lax.axis_index("s") * 2 + lax.axis_index("c")
        sl = pl.ds(wid * per_w, per_w)

        def one(idx_hbm, tab_hbm, out_hbm):
            pltpu.sync_copy(idx_hbm.at[sl], idx_v)
            pltpu.async_copy(tab_hbm.at[idx_v], rows_v, sem).wait()
            pltpu.sync_copy(rows_v, out_hbm.at[sl])

        one(sa, c1a, o1sa)
        one(sb, c1b, o1sb)
        one(ea, c1a, o1ea)
        one(eb, c1b, o1eb)
        one(sa, c4a, o4sa)
        one(sb, c4b, o4sb)
        one(ea, c4a, o4ea)
        one(eb, c4b, o4eb)

    return k(C1a, C1b, C4a, C4b, ia_s, ib_s, ia_e, ib_e)


def _cumsum_kernel(x1_ref, x4_ref, c1_ref, c4_ref, carry_ref):
    @pl.when(pl.program_id(0) == 0)
    def _():
        carry_ref[...] = jnp.zeros_like(carry_ref)

    cs1 = _inclusive_cumsum(x1_ref[...]) + carry_ref[0:1, :]
    c1_ref[...] = cs1
    carry_ref[0:1, :] = cs1[-1:, :]
    cs4 = _inclusive_cumsum(x4_ref[...]) + carry_ref[1:2, :]
    c4_ref[...] = cs4
    carry_ref[1:2, :] = cs4[-1:, :]


def _cumsum2(x1, x4):
    n = x1.shape[0]
    return pl.pallas_call(
        _cumsum_kernel,
        grid=(n // _EBLK,),
        in_specs=[
            pl.BlockSpec((_EBLK, _D), lambda i: (i, 0)),
            pl.BlockSpec((_EBLK, _D), lambda i: (i, 0)),
        ],
        out_specs=[
            pl.BlockSpec((_EBLK, _D), lambda i: (i, 0)),
            pl.BlockSpec((_EBLK, _D), lambda i: (i, 0)),
        ],
        out_shape=[
            jax.ShapeDtypeStruct((n, _D), jnp.float32),
            jax.ShapeDtypeStruct((n, _D), jnp.float32),
        ],
        scratch_shapes=[pltpu.VMEM((8, _D), jnp.float32)],
    )(x1, x4)


def _edge_kernel(a_ref, b_ref, ve_ref, W1_ref, b1_ref, W2p_ref, b2p_ref,
                 We1_ref, We2_ref, be_ref, f1_ref, f2_ref, f3_ref, f4_ref,
                 vepre_ref, stats_ref):
    x = jnp.concatenate([a_ref[...], b_ref[...], ve_ref[...]], axis=1)
    h = jnp.maximum(x @ W1_ref[...] + b1_ref[...], 0.0)
    m = h @ W2p_ref[...] + b2p_ref[...]
    k = jax.nn.sigmoid(m[:, 4 * _D:4 * _D + 1])
    f = m[:, :4 * _D] * k
    f1_ref[...] = f[:, :_D]
    f2_ref[...] = f[:, _D:2 * _D]
    f3_ref[...] = f[:, 2 * _D:3 * _D]
    f4_ref[...] = f[:, 3 * _D:]
    vp = f @ We1_ref[...] + ve_ref[...] @ We2_ref[...] + be_ref[...]
    vepre_ref[...] = vp

    @pl.when(pl.program_id(0) == 0)
    def _():
        stats_ref[...] = jnp.zeros_like(stats_ref)

    s = jnp.sum(vp, axis=0, keepdims=True)
    s2 = jnp.sum(vp * vp, axis=0, keepdims=True)
    stats_ref[0:1, :] += s
    stats_ref[1:2, :] += s2


def _node_kernel(vc_ref, nf1_ref, nf2_ref, nf3_ref, nf4_ref, Wr_ref, br_ref,
                 out_ref, stats_ref):
    Wr = Wr_ref[...]
    o = (vc_ref[...] @ Wr[0:_D] + nf1_ref[...] @ Wr[_D:2 * _D]
         + nf2_ref[...] @ Wr[2 * _D:3 * _D] + nf3_ref[...] @ Wr[3 * _D:4 * _D]
         + nf4_ref[...] @ Wr[4 * _D:5 * _D] + br_ref[...])
    out_ref[...] = o

    @pl.when(pl.program_id(0) == 0)
    def _():
        stats_ref[...] = jnp.zeros_like(stats_ref)

    stats_ref[0:1, :] += jnp.sum(o, axis=0, keepdims=True)
    stats_ref[1:2, :] += jnp.sum(o * o, axis=0, keepdims=True)


def _norm_kernel(x_ref, scale_ref, shift_ref, o_ref):
    o_ref[...] = x_ref[...] * scale_ref[...] + shift_ref[...]


def _normalize(x, scale, shift, blk):
    n = x.shape[0]
    return pl.pallas_call(
        _norm_kernel,
        grid=(n // blk,),
        in_specs=[
            pl.BlockSpec((blk, _D), lambda i: (i, 0)),
            pl.BlockSpec((1, _D), lambda i: (0, 0)),
            pl.BlockSpec((1, _D), lambda i: (0, 0)),
        ],
        out_specs=pl.BlockSpec((blk, _D), lambda i: (i, 0)),
        out_shape=jax.ShapeDtypeStruct((n, _D), jnp.float32),
    )(x, scale, shift)


def kernel(in_vc, in_ve, edge_index, W1, b1, W2, b2, Wr, br, We, be,
           gamma_gc, beta_gc, gamma_ef, beta_ef):
    src = edge_index[0]
    dst = edge_index[1]

    W2p = jnp.concatenate(
        [W2[:, 1:], W2[:, 0:1], jnp.zeros((3 * _D, _D - 1), jnp.float32)], axis=1)
    b2p = jnp.concatenate(
        [b2[1:], b2[0:1], jnp.zeros((_D - 1,), jnp.float32)])[None, :]
    b1r = b1[None, :]
    ber = be[None, :]
    brr = br[None, :]
    We1 = We[:4 * _D]
    We2 = We[4 * _D:]

    sdst, perm = lax.sort_key_val(dst, lax.iota(jnp.int32, _E))

    a, b, veg = _sc_gather_sorted(in_vc, in_ve, src, sdst, perm)

    deg = jax.ops.segment_sum(jnp.ones((_E,), jnp.float32), dst,
                              num_segments=_N)

    half = _E // 2
    nhb = half // _EBLK

    def _edge_half(off):
        return pl.pallas_call(
            _edge_kernel,
            grid=(nhb,),
            in_specs=[
                pl.BlockSpec((_EBLK, _D), lambda i: (i + off, 0)),
                pl.BlockSpec((_EBLK, _D), lambda i: (i + off, 0)),
                pl.BlockSpec((_EBLK, _D), lambda i: (i + off, 0)),
                pl.BlockSpec((3 * _D, 3 * _D), lambda i: (0, 0)),
                pl.BlockSpec((1, 3 * _D), lambda i: (0, 0)),
                pl.BlockSpec((3 * _D, 5 * _D), lambda i: (0, 0)),
                pl.BlockSpec((1, 5 * _D), lambda i: (0, 0)),
                pl.BlockSpec((4 * _D, _D), lambda i: (0, 0)),
                pl.BlockSpec((_D, _D), lambda i: (0, 0)),
                pl.BlockSpec((1, _D), lambda i: (0, 0)),
            ],
            out_specs=[
                pl.BlockSpec((_EBLK, _D), lambda i: (i, 0)),
                pl.BlockSpec((_EBLK, _D), lambda i: (i, 0)),
                pl.BlockSpec((_EBLK, _D), lambda i: (i, 0)),
                pl.BlockSpec((_EBLK, _D), lambda i: (i, 0)),
                pl.BlockSpec((_EBLK, _D), lambda i: (i, 0)),
                pl.BlockSpec((8, _D), lambda i: (0, 0)),
            ],
            out_shape=[
                jax.ShapeDtypeStruct((half, _D), jnp.float32),
                jax.ShapeDtypeStruct((half, _D), jnp.float32),
                jax.ShapeDtypeStruct((half, _D), jnp.float32),
                jax.ShapeDtypeStruct((half, _D), jnp.float32),
                jax.ShapeDtypeStruct((half, _D), jnp.float32),
                jax.ShapeDtypeStruct((8, _D), jnp.float32),
            ],
        )(a, b, veg, W1, b1r, W2p, b2p, We1, We2, ber)

    sdst1 = sdst[:half]
    sdst2 = sdst[half:]
    f1a, f2a, f3a, f4a, vpa, esta = _edge_half(0)
    C1a, C4a = _cumsum2(f1a, f4a)
    m2a = jax.ops.segment_max(f2a, sdst1, num_segments=_N,
                              indices_are_sorted=True)
    m3a = jax.ops.segment_min(f3a, sdst1, num_segments=_N,
                              indices_are_sorted=True)
    f1b, f2b, f3b, f4b, vpb, estb = _edge_half(nhb)
    C1b, C4b = _cumsum2(f1b, f4b)
    m2b = jax.ops.segment_max(f2b, sdst2, num_segments=_N,
                              indices_are_sorted=True)
    m3b = jax.ops.segment_min(f3b, sdst2, num_segments=_N,
                              indices_are_sorted=True)

    rp_end = jnp.cumsum(deg).astype(jnp.int32)
    rp_start = rp_end - deg.astype(jnp.int32)

    T1a = C1a[half - 1]
    T4a = C4a[half - 1]

    pad = jnp.zeros((_NPAD - _N,), jnp.int32)

    def _pidx(idx):
        ia = jnp.clip(idx - 1, 0, half - 1)
        ib = jnp.clip(idx - half - 1, 0, half - 1)
        return (jnp.concatenate([ia, pad]), jnp.concatenate([ib, pad]))

    ia_s, ib_s = _pidx(rp_start)
    ia_e, ib_e = _pidx(rp_end)
    g = _sc_boundary_gather(C1a, C1b, C4a, C4b, ia_s, ib_s, ia_e, ib_e)
    g = [x[:_N] for x in g]

    def _cum_at(va, vb, Ta, idx):
        v = jnp.where((idx > half)[:, None], vb + Ta[None, :], va)
        return jnp.where((idx == 0)[:, None], 0.0, v)

    nf1 = (_cum_at(g[2], g[3], T1a, rp_end)
           - _cum_at(g[0], g[1], T1a, rp_start))
    s4 = (_cum_at(g[6], g[7], T4a, rp_end)
          - _cum_at(g[4], g[5], T4a, rp_start))

    vepre = jnp.concatenate([vpa, vpb], axis=0)
    est = esta + estb

    has = (deg > 0)[:, None]
    nf2 = jnp.where(has, jnp.maximum(m2a, m2b), 0.0)
    nf3 = jnp.where(has, jnp.minimum(m3a, m3b), 0.0)
    nf4 = s4 / jnp.maximum(deg, 1.0)[:, None]

    vcpre, nst = pl.pallas_call(
        _node_kernel,
        grid=(_N // _NBLK,),
        in_specs=[
            pl.BlockSpec((_NBLK, _D), lambda i: (i, 0)),
            pl.BlockSpec((_NBLK, _D), lambda i: (i, 0)),
            pl.BlockSpec((_NBLK, _D), lambda i: (i, 0)),
            pl.BlockSpec((_NBLK, _D), lambda i: (i, 0)),
            pl.BlockSpec((_NBLK, _D), lambda i: (i, 0)),
            pl.BlockSpec((5 * _D, _D), lambda i: (0, 0)),
            pl.BlockSpec((1, _D), lambda i: (0, 0)),
        ],
        out_specs=[
            pl.BlockSpec((_NBLK, _D), lambda i: (i, 0)),
            pl.BlockSpec((8, _D), lambda i: (0, 0)),
        ],
        out_shape=[
            jax.ShapeDtypeStruct((_N, _D), jnp.float32),
            jax.ShapeDtypeStruct((8, _D), jnp.float32),
        ],
    )(in_vc, nf1, nf2, nf3, nf4, Wr, brr)

    def _bn_coeffs(stats, n, gamma, beta):
        mu = stats[0] / n
        var = stats[1] / n - mu * mu
        rstd = jax.lax.rsqrt(var + _EPS)
        scale = gamma * rstd
        shift = beta - mu * scale
        return scale[None, :], shift[None, :]

    esc, esh = _bn_coeffs(est, float(_E), gamma_ef, beta_ef)
    nsc, nsh = _bn_coeffs(nst, float(_N), gamma_gc, beta_gc)

    out_vc = _normalize(vcpre, nsc, nsh, _NBLK)
    out_ve = _sc_unpermute(_normalize(vepre, esc, esh, 2000), perm)
    return (out_vc, out_ve)


# speedup vs baseline: 1.4598x; 1.0134x over previous
import functools
import jax
import jax.numpy as jnp
from jax import lax
from jax.experimental import pallas as pl
from jax.experimental.pallas import tpu as pltpu
from jax.experimental.pallas import tpu_sc as plsc

_D = 128
_N = 10000
_E = 320000
_EPS = 1e-5
_EBLK = 640
_NBLK = 1000

_NW = 32
_GCHUNK = 400


def _sc_gather_sorted(table, ve, src, sdst, perm):
    per_w = _E // _NW
    chunk = 200
    n_it = per_w // chunk
    mesh = plsc.VectorSubcoreMesh(core_axis_name="c", subcore_axis_name="s")

    @functools.partial(
        pl.kernel,
        out_type=[jax.ShapeDtypeStruct((_E, _D), jnp.float32),
                  jax.ShapeDtypeStruct((_E, _D), jnp.float32),
                  jax.ShapeDtypeStruct((_E, _D), jnp.float32)],
        mesh=mesh,
        scratch_types=[
            pltpu.VMEM((chunk,), jnp.int32),
            pltpu.VMEM((chunk,), jnp.int32),
            pltpu.VMEM((chunk,), jnp.int32),
            pltpu.VMEM((chunk, _D), jnp.float32),
            pltpu.VMEM((chunk, _D), jnp.float32),
            pltpu.VMEM((chunk, _D), jnp.float32),
            pltpu.SemaphoreType.DMA,
            pltpu.SemaphoreType.DMA,
            pltpu.SemaphoreType.DMA,
        ],
    )
    def k(table_hbm, ve_hbm, src_hbm, sdst_hbm, perm_hbm,
          a_hbm, b_hbm, veg_hbm, perm_v, sdst_v, srcg_v,
          ra_v, rb_v, rv_v, sa, sb, sv):
        wid = lax.axis_index("s") * 2 + lax.axis_index("c")
        base0 = wid * per_w

        def body(j, carry):
            base = base0 + j * chunk
            sl = pl.ds(base, chunk)
            pltpu.sync_copy(perm_hbm.at[sl], perm_v)
            pltpu.sync_copy(sdst_hbm.at[sl], sdst_v)
            gb = pltpu.async_copy(table_hbm.at[sdst_v], rb_v, sb)
            gv = pltpu.async_copy(ve_hbm.at[perm_v], rv_v, sv)
            pltpu.async_copy(src_hbm.at[perm_v], srcg_v, sa).wait()
            ga = pltpu.async_copy(table_hbm.at[srcg_v], ra_v, sa)
            gb.wait()
            wb = pltpu.async_copy(rb_v, b_hbm.at[sl], sb)
            gv.wait()
            wv = pltpu.async_copy(rv_v, veg_hbm.at[sl], sv)
            ga.wait()
            wa = pltpu.async_copy(ra_v, a_hbm.at[sl], sa)
            wb.wait()
            wv.wait()
            wa.wait()
            return carry

        lax.fori_loop(0, n_it, body, 0)

    return k(table, ve, src, sdst, perm)


def _sc_unpermute(x, perm):
    per_w = _E // _NW
    n_it = per_w // _GCHUNK
    mesh = plsc.VectorSubcoreMesh(core_axis_name="c", subcore_axis_name="s")

    @functools.partial(
        pl.kernel,
        out_type=jax.ShapeDtypeStruct((_E, _D), jnp.float32),
        mesh=mesh,
        scratch_types=[
            pltpu.VMEM((_GCHUNK,), jnp.int32),
            pltpu.VMEM((_GCHUNK, _D), jnp.float32),
            pltpu.SemaphoreType.DMA,
        ],
    )
    def k(x_hbm, perm_hbm, out_hbm, perm_v, rows_v, sem):
        wid = lax.axis_index("s") * 2 + lax.axis_index("c")
        base0 = wid * per_w

        def body(j, carry):
            base = base0 + j * _GCHUNK
            sl = pl.ds(base, _GCHUNK)
            pltpu.sync_copy(perm_hbm.at[sl], perm_v)
            pltpu.sync_copy(x_hbm.at[sl], rows_v)
            pltpu.async_copy(rows_v, out_hbm.at[perm_v], sem).wait()
            return carry

        lax.fori_loop(0, n_it, body, 0)

    return k(x, perm)


def _inclusive_cumsum(x):
    n = x.shape[0]
    s = x
    shift = 1
    while shift < n:
        z = jnp.zeros((shift, x.shape[1]), x.dtype)
        s = s + jnp.concatenate([z, s[:-shift]], axis=0)
        shift *= 2
    return s


_NPAD = 10240


def _sc_boundary_gather(C1a, C1b, C4a, C4b, ia_s, ib_s, ia_e, ib_e):
    per_w = _NPAD // _NW
    mesh = plsc.VectorSubcoreMesh(core_axis_name="c", subcore_axis_name="s")
    ot = jax.ShapeDtypeStruct((_NPAD, _D), jnp.float32)

    @functools.partial(
        pl.kernel,
        out_type=[ot] * 8,
        mesh=mesh,
        scratch_types=[
            pltpu.VMEM((per_w,), jnp.int32),
            pltpu.VMEM((per_w, _D), jnp.float32),
            pltpu.SemaphoreType.DMA,
        ],
    )
    def k(c1a, c1b, c4a, c4b, sa, sb, ea, eb,
          o1sa, o1sb, o1ea, o1eb, o4sa, o4sb, o4ea, o4eb, idx_v, rows_v, sem):
        wid = lax.axis_index("s") * 2 + lax.axis_index("c")
        sl = pl.ds(wid * per_w, per_w)

        def one(idx_hbm, tab_hbm, out_hbm):
            pltpu.sync_copy(idx_hbm.at[sl], idx_v)
            pltpu.async_copy(tab_hbm.at[idx_v], rows_v, sem).wait()
            pltpu.sync_copy(rows_v, out_hbm.at[sl])

        one(sa, c1a, o1sa)
        one(sb, c1b, o1sb)
        one(ea, c1a, o1ea)
        one(eb, c1b, o1eb)
        one(sa, c4a, o4sa)
        one(sb, c4b, o4sb)
        one(ea, c4a, o4ea)
        one(eb, c4b, o4eb)

    return k(C1a, C1b, C4a, C4b, ia_s, ib_s, ia_e, ib_e)


def _cumsum_kernel(x1_ref, x4_ref, c1_ref, c4_ref, carry_ref):
    @pl.when(pl.program_id(0) == 0)
    def _():
        carry_ref[...] = jnp.zeros_like(carry_ref)

    cs1 = _inclusive_cumsum(x1_ref[...]) + carry_ref[0:1, :]
    c1_ref[...] = cs1
    carry_ref[0:1, :] = cs1[-1:, :]
    cs4 = _inclusive_cumsum(x4_ref[...]) + carry_ref[1:2, :]
    c4_ref[...] = cs4
    carry_ref[1:2, :] = cs4[-1:, :]


def _cumsum2(x1, x4):
    n = x1.shape[0]
    return pl.pallas_call(
        _cumsum_kernel,
        grid=(n // _EBLK,),
        in_specs=[
            pl.BlockSpec((_EBLK, _D), lambda i: (i, 0)),
            pl.BlockSpec((_EBLK, _D), lambda i: (i, 0)),
        ],
        out_specs=[
            pl.BlockSpec((_EBLK, _D), lambda i: (i, 0)),
            pl.BlockSpec((_EBLK, _D), lambda i: (i, 0)),
        ],
        out_shape=[
            jax.ShapeDtypeStruct((n, _D), jnp.float32),
            jax.ShapeDtypeStruct((n, _D), jnp.float32),
        ],
        scratch_shapes=[pltpu.VMEM((8, _D), jnp.float32)],
    )(x1, x4)


def _edge_kernel(a_ref, b_ref, ve_ref, W1_ref, b1_ref, W2p_ref, b2p_ref,
                 We1_ref, We2_ref, be_ref, f1_ref, f2_ref, f3_ref, f4_ref,
                 vepre_ref, stats_ref):
    x = jnp.concatenate([a_ref[...], b_ref[...], ve_ref[...]], axis=1)
    h = jnp.maximum(x @ W1_ref[...] + b1_ref[...], 0.0)
    m = h @ W2p_ref[...] + b2p_ref[...]
    k = jax.nn.sigmoid(m[:, 4 * _D:4 * _D + 1])
    f = m[:, :4 * _D] * k
    f1_ref[...] = f[:, :_D]
    f2_ref[...] = f[:, _D:2 * _D]
    f3_ref[...] = f[:, 2 * _D:3 * _D]
    f4_ref[...] = f[:, 3 * _D:]
    vp = f @ We1_ref[...] + ve_ref[...] @ We2_ref[...] + be_ref[...]
    vepre_ref[...] = vp

    @pl.when(pl.program_id(0) == 0)
    def _():
        stats_ref[...] = jnp.zeros_like(stats_ref)

    s = jnp.sum(vp, axis=0, keepdims=True)
    s2 = jnp.sum(vp * vp, axis=0, keepdims=True)
    stats_ref[0:1, :] += s
    stats_ref[1:2, :] += s2


def _node_kernel(vc_ref, nf1_ref, nf2_ref, nf3_ref, nf4_ref, Wr_ref, br_ref,
                 out_ref, stats_ref):
    Wr = Wr_ref[...]
    o = (vc_ref[...] @ Wr[0:_D] + nf1_ref[...] @ Wr[_D:2 * _D]
         + nf2_ref[...] @ Wr[2 * _D:3 * _D] + nf3_ref[...] @ Wr[3 * _D:4 * _D]
         + nf4_ref[...] @ Wr[4 * _D:5 * _D] + br_ref[...])
    out_ref[...] = o

    @pl.when(pl.program_id(0) == 0)
    def _():
        stats_ref[...] = jnp.zeros_like(stats_ref)

    stats_ref[0:1, :] += jnp.sum(o, axis=0, keepdims=True)
    stats_ref[1:2, :] += jnp.sum(o * o, axis=0, keepdims=True)


def _norm_kernel(x_ref, scale_ref, shift_ref, o_ref):
    o_ref[...] = x_ref[...] * scale_ref[...] + shift_ref[...]


def _normalize(x, scale, shift, blk):
    n = x.shape[0]
    return pl.pallas_call(
        _norm_kernel,
        grid=(n // blk,),
        in_specs=[
            pl.BlockSpec((blk, _D), lambda i: (i, 0)),
            pl.BlockSpec((1, _D), lambda i: (0, 0)),
            pl.BlockSpec((1, _D), lambda i: (0, 0)),
        ],
        out_specs=pl.BlockSpec((blk, _D), lambda i: (i, 0)),
        out_shape=jax.ShapeDtypeStruct((n, _D), jnp.float32),
    )(x, scale, shift)


def kernel(in_vc, in_ve, edge_index, W1, b1, W2, b2, Wr, br, We, be,
           gamma_gc, beta_gc, gamma_ef, beta_ef):
    src = edge_index[0]
    dst = edge_index[1]

    W2p = jnp.concatenate(
        [W2[:, 1:], W2[:, 0:1], jnp.zeros((3 * _D, _D - 1), jnp.float32)], axis=1)
    b2p = jnp.concatenate(
        [b2[1:], b2[0:1], jnp.zeros((_D - 1,), jnp.float32)])[None, :]
    b1r = b1[None, :]
    ber = be[None, :]
    brr = br[None, :]
    We1 = We[:4 * _D]
    We2 = We[4 * _D:]

    sdst, perm = lax.sort_key_val(dst, lax.iota(jnp.int32, _E))

    a, b, veg = _sc_gather_sorted(in_vc, in_ve, src, sdst, perm)

    deg = jax.ops.segment_sum(jnp.ones((_E,), jnp.float32), dst,
                              num_segments=_N)

    half = _E // 2
    nhb = half // _EBLK

    def _edge_half(off):
        return pl.pallas_call(
            _edge_kernel,
            grid=(nhb,),
            in_specs=[
                pl.BlockSpec((_EBLK, _D), lambda i: (i + off, 0)),
                pl.BlockSpec((_EBLK, _D), lambda i: (i + off, 0)),
                pl.BlockSpec((_EBLK, _D), lambda i: (i + off, 0)),
                pl.BlockSpec((3 * _D, 3 * _D), lambda i: (0, 0)),
                pl.BlockSpec((1, 3 * _D), lambda i: (0, 0)),
                pl.BlockSpec((3 * _D, 5 * _D), lambda i: (0, 0)),
                pl.BlockSpec((1, 5 * _D), lambda i: (0, 0)),
                pl.BlockSpec((4 * _D, _D), lambda i: (0, 0)),
                pl.BlockSpec((_D, _D), lambda i: (0, 0)),
                pl.BlockSpec((1, _D), lambda i: (0, 0)),
            ],
            out_specs=[
                pl.BlockSpec((_EBLK, _D), lambda i: (i, 0)),
                pl.BlockSpec((_EBLK, _D), lambda i: (i, 0)),
                pl.BlockSpec((_EBLK, _D), lambda i: (i, 0)),
                pl.BlockSpec((_EBLK, _D), lambda i: (i, 0)),
                pl.BlockSpec((_EBLK, _D), lambda i: (i, 0)),
                pl.BlockSpec((8, _D), lambda i: (0, 0)),
            ],
            out_shape=[
                jax.ShapeDtypeStruct((half, _D), jnp.float32),
                jax.ShapeDtypeStruct((half, _D), jnp.float32),
                jax.ShapeDtypeStruct((half, _D), jnp.float32),
                jax.ShapeDtypeStruct((half, _D), jnp.float32),
                jax.ShapeDtypeStruct((half, _D), jnp.float32),
                jax.ShapeDtypeStruct((8, _D), jnp.float32),
            ],
        )(a, b, veg, W1, b1r, W2p, b2p, We1, We2, ber)

    sdst1 = sdst[:half]
    sdst2 = sdst[half:]
    f1a, f2a, f3a, f4a, vpa, esta = _edge_half(0)
    C1a, C4a = _cumsum2(f1a, f4a)
    m2a = jax.ops.segment_max(f2a, sdst1, num_segments=_N,
                              indices_are_sorted=True)
    m3a = jax.ops.segment_min(f3a, sdst1, num_segments=_N,
                              indices_are_sorted=True)
    f1b, f2b, f3b, f4b, vpb, estb = _edge_half(nhb)
    C1b, C4b = _cumsum2(f1b, f4b)
    m2b = jax.ops.segment_max(f2b, sdst2, num_segments=_N,
                              indices_are_sorted=True)
    m3b = jax.ops.segment_min(f3b, sdst2, num_segments=_N,
                              indices_are_sorted=True)

    rp_end = jnp.cumsum(deg).astype(jnp.int32)
    rp_start = rp_end - deg.astype(jnp.int32)

    T1a = C1a[half - 1]
    T4a = C4a[half - 1]

    pad = jnp.zeros((_NPAD - _N,), jnp.int32)

    def _pidx(idx):
        ia = jnp.clip(idx - 1, 0, half - 1)
        ib = jnp.clip(idx - half - 1, 0, half - 1)
        return (jnp.concatenate([ia, pad]), jnp.concatenate([ib, pad]))

    ia_s, ib_s = _pidx(rp_start)
    ia_e, ib_e = _pidx(rp_end)
    g = _sc_boundary_gather(C1a, C1b, C4a, C4b, ia_s, ib_s, ia_e, ib_e)
    g = [x[:_N] for x in g]

    def _cum_at(va, vb, Ta, idx):
        v = jnp.where((idx > half)[:, None], vb + Ta[None, :], va)
        return jnp.where((idx == 0)[:, None], 0.0, v)

    nf1 = (_cum_at(g[2], g[3], T1a, rp_end)
           - _cum_at(g[0], g[1], T1a, rp_start))
    s4 = (_cum_at(g[6], g[7], T4a, rp_end)
          - _cum_at(g[4], g[5], T4a, rp_start))

    vepre = jnp.concatenate([vpa, vpb], axis=0)
    est = esta + estb

    has = (deg > 0)[:, None]
    nf2 = jnp.where(has, jnp.maximum(m2a, m2b), 0.0)
    nf3 = jnp.where(has, jnp.minimum(m3a, m3b), 0.0)
    nf4 = s4 / jnp.maximum(deg, 1.0)[:, None]

    vcpre, nst = pl.pallas_call(
        _node_kernel,
        grid=(_N // _NBLK,),
        in_specs=[
            pl.BlockSpec((_NBLK, _D), lambda i: (i, 0)),
            pl.BlockSpec((_NBLK, _D), lambda i: (i, 0)),
            pl.BlockSpec((_NBLK, _D), lambda i: (i, 0)),
            pl.BlockSpec((_NBLK, _D), lambda i: (i, 0)),
            pl.BlockSpec((_NBLK, _D), lambda i: (i, 0)),
            pl.BlockSpec((5 * _D, _D), lambda i: (0, 0)),
            pl.BlockSpec((1, _D), lambda i: (0, 0)),
        ],
        out_specs=[
            pl.BlockSpec((_NBLK, _D), lambda i: (i, 0)),
            pl.BlockSpec((8, _D), lambda i: (0, 0)),
        ],
        out_shape=[
            jax.ShapeDtypeStruct((_N, _D), jnp.float32),
            jax.ShapeDtypeStruct((8, _D), jnp.float32),
        ],
    )(in_vc, nf1, nf2, nf3, nf4, Wr, brr)

    def _bn_coeffs(stats, n, gamma, beta):
        mu = stats[0] / n
        var = stats[1] / n - mu * mu
        rstd = jax.lax.rsqrt(var + _EPS)
        scale = gamma * rstd
        shift = beta - mu * scale
        return scale[None, :], shift[None, :]

    esc, esh = _bn_coeffs(est, float(_E), gamma_ef, beta_ef)
    nsc, nsh = _bn_coeffs(nst, float(_N), gamma_gc, beta_gc)

    out_vc = _normalize(vcpre, nsc, nsh, _NBLK)
    out_ve = _sc_unpermute(_normalize(vepre, esc, esh, 2000), perm)
    return (out_vc, out_ve)


# speedup vs baseline: 1.6958x; 1.1616x over previous
import functools
import jax
import jax.numpy as jnp
from jax import lax
from jax.experimental import pallas as pl
from jax.experimental.pallas import tpu as pltpu
from jax.experimental.pallas import tpu_sc as plsc

_D = 128
_N = 10000
_E = 320000
_EPS = 1e-5
_EBLK = 640
_NBLK = 1000

_NW = 32
_GCHUNK = 400


def _sc_gather_sorted(table, ve, src, sdst, perm):
    per_w = _E // _NW
    chunk = 200
    n_it = per_w // chunk
    mesh = plsc.VectorSubcoreMesh(core_axis_name="c", subcore_axis_name="s")

    @functools.partial(
        pl.kernel,
        out_type=[jax.ShapeDtypeStruct((_E, _D), jnp.float32),
                  jax.ShapeDtypeStruct((_E, _D), jnp.float32),
                  jax.ShapeDtypeStruct((_E, _D), jnp.float32)],
        mesh=mesh,
        scratch_types=[
            pltpu.VMEM((chunk,), jnp.int32),
            pltpu.VMEM((chunk,), jnp.int32),
            pltpu.VMEM((chunk,), jnp.int32),
            pltpu.VMEM((chunk, _D), jnp.float32),
            pltpu.VMEM((chunk, _D), jnp.float32),
            pltpu.VMEM((chunk, _D), jnp.float32),
            pltpu.SemaphoreType.DMA,
            pltpu.SemaphoreType.DMA,
            pltpu.SemaphoreType.DMA,
        ],
    )
    def k(table_hbm, ve_hbm, src_hbm, sdst_hbm, perm_hbm,
          a_hbm, b_hbm, veg_hbm, perm_v, sdst_v, srcg_v,
          ra_v, rb_v, rv_v, sa, sb, sv):
        wid = lax.axis_index("s") * 2 + lax.axis_index("c")
        base0 = wid * per_w

        def body(j, carry):
            base = base0 + j * chunk
            sl = pl.ds(base, chunk)
            pltpu.sync_copy(perm_hbm.at[sl], perm_v)
            pltpu.sync_copy(sdst_hbm.at[sl], sdst_v)
            gb = pltpu.async_copy(table_hbm.at[sdst_v], rb_v, sb)
            gv = pltpu.async_copy(ve_hbm.at[perm_v], rv_v, sv)
            pltpu.async_copy(src_hbm.at[perm_v], srcg_v, sa).wait()
            ga = pltpu.async_copy(table_hbm.at[srcg_v], ra_v, sa)
            gb.wait()
            wb = pltpu.async_copy(rb_v, b_hbm.at[sl], sb)
            gv.wait()
            wv = pltpu.async_copy(rv_v, veg_hbm.at[sl], sv)
            ga.wait()
            wa = pltpu.async_copy(ra_v, a_hbm.at[sl], sa)
            wb.wait()
            wv.wait()
            wa.wait()
            return carry

        lax.fori_loop(0, n_it, body, 0)

    return k(table, ve, src, sdst, perm)


def _sc_unpermute(x, perm):
    per_w = _E // _NW
    n_it = per_w // _GCHUNK
    mesh = plsc.VectorSubcoreMesh(core_axis_name="c", subcore_axis_name="s")

    @functools.partial(
        pl.kernel,
        out_type=jax.ShapeDtypeStruct((_E, _D), jnp.float32),
        mesh=mesh,
        scratch_types=[
            pltpu.VMEM((_GCHUNK,), jnp.int32),
            pltpu.VMEM((_GCHUNK, _D), jnp.float32),
            pltpu.SemaphoreType.DMA,
        ],
    )
    def k(x_hbm, perm_hbm, out_hbm, perm_v, rows_v, sem):
        wid = lax.axis_index("s") * 2 + lax.axis_index("c")
        base0 = wid * per_w

        def body(j, carry):
            base = base0 + j * _GCHUNK
            sl = pl.ds(base, _GCHUNK)
            pltpu.sync_copy(perm_hbm.at[sl], perm_v)
            pltpu.sync_copy(x_hbm.at[sl], rows_v)
            pltpu.async_copy(rows_v, out_hbm.at[perm_v], sem).wait()
            return carry

        lax.fori_loop(0, n_it, body, 0)

    return k(x, perm)


def _inclusive_cumsum(x):
    n = x.shape[0]
    s = x
    shift = 1
    while shift < n:
        z = jnp.zeros((shift, x.shape[1]), x.dtype)
        s = s + jnp.concatenate([z, s[:-shift]], axis=0)
        shift *= 2
    return s


_NPAD = 10240


def _sc_boundary_gather(C1a, C1b, C4a, C4b, ia, ib):
    per_w = _NPAD // _NW
    chunk = 160
    mesh = plsc.VectorSubcoreMesh(core_axis_name="c", subcore_axis_name="s")
    ot = jax.ShapeDtypeStruct((_NPAD, _D), jnp.float32)

    @functools.partial(
        pl.kernel,
        out_type=[ot] * 4,
        mesh=mesh,
        scratch_types=[
            pltpu.VMEM((chunk,), jnp.int32),
            pltpu.VMEM((chunk,), jnp.int32),
            pltpu.VMEM((chunk, _D), jnp.float32),
            pltpu.VMEM((chunk, _D), jnp.float32),
            pltpu.VMEM((chunk, _D), jnp.float32),
            pltpu.VMEM((chunk, _D), jnp.float32),
            pltpu.SemaphoreType.DMA,
            pltpu.SemaphoreType.DMA,
            pltpu.SemaphoreType.DMA,
            pltpu.SemaphoreType.DMA,
        ],
    )
    def k(c1a, c1b, c4a, c4b, iah, ibh,
          o1a, o1b, o4a, o4b, ia_v, ib_v, r1a, r1b, r4a, r4b,
          s1, s2, s3, s4):
        wid = lax.axis_index("s") * 2 + lax.axis_index("c")
        base0 = wid * per_w

        def body(j, carry):
            sl = pl.ds(base0 + j * chunk, chunk)
            pltpu.sync_copy(iah.at[sl], ia_v)
            pltpu.sync_copy(ibh.at[sl], ib_v)
            g1 = pltpu.async_copy(c1a.at[ia_v], r1a, s1)
            g2 = pltpu.async_copy(c1b.at[ib_v], r1b, s2)
            g3 = pltpu.async_copy(c4a.at[ia_v], r4a, s3)
            g4 = pltpu.async_copy(c4b.at[ib_v], r4b, s4)
            g1.wait()
            w1 = pltpu.async_copy(r1a, o1a.at[sl], s1)
            g2.wait()
            w2 = pltpu.async_copy(r1b, o1b.at[sl], s2)
            g3.wait()
            w3 = pltpu.async_copy(r4a, o4a.at[sl], s3)
            g4.wait()
            w4 = pltpu.async_copy(r4b, o4b.at[sl], s4)
            w1.wait()
            w2.wait()
            w3.wait()
            w4.wait()
            return carry

        lax.fori_loop(0, per_w // chunk, body, 0)

    return k(C1a, C1b, C4a, C4b, ia, ib)


def _cumsum_kernel(x1_ref, x4_ref, c1_ref, c4_ref, carry_ref):
    @pl.when(pl.program_id(0) == 0)
    def _():
        carry_ref[...] = jnp.zeros_like(carry_ref)

    cs1 = _inclusive_cumsum(x1_ref[...]) + carry_ref[0:1, :]
    c1_ref[...] = cs1
    carry_ref[0:1, :] = cs1[-1:, :]
    cs4 = _inclusive_cumsum(x4_ref[...]) + carry_ref[1:2, :]
    c4_ref[...] = cs4
    carry_ref[1:2, :] = cs4[-1:, :]


def _cumsum2(x1, x4):
    n = x1.shape[0]
    return pl.pallas_call(
        _cumsum_kernel,
        grid=(n // _EBLK,),
        in_specs=[
            pl.BlockSpec((_EBLK, _D), lambda i: (i, 0)),
            pl.BlockSpec((_EBLK, _D), lambda i: (i, 0)),
        ],
        out_specs=[
            pl.BlockSpec((_EBLK, _D), lambda i: (i, 0)),
            pl.BlockSpec((_EBLK, _D), lambda i: (i, 0)),
        ],
        out_shape=[
            jax.ShapeDtypeStruct((n, _D), jnp.float32),
            jax.ShapeDtypeStruct((n, _D), jnp.float32),
        ],
        scratch_shapes=[pltpu.VMEM((8, _D), jnp.float32)],
    )(x1, x4)


def _edge_kernel(a_ref, b_ref, ve_ref, W1_ref, b1_ref, W2p_ref, b2p_ref,
                 We1_ref, We2_ref, be_ref, f1_ref, f2_ref, f3_ref, f4_ref,
                 vepre_ref, stats_ref):
    x = jnp.concatenate([a_ref[...], b_ref[...], ve_ref[...]], axis=1)
    h = jnp.maximum(x @ W1_ref[...] + b1_ref[...], 0.0)
    m = h @ W2p_ref[...] + b2p_ref[...]
    k = jax.nn.sigmoid(m[:, 4 * _D:4 * _D + 1])
    f = m[:, :4 * _D] * k
    f1_ref[...] = f[:, :_D]
    f2_ref[...] = f[:, _D:2 * _D]
    f3_ref[...] = f[:, 2 * _D:3 * _D]
    f4_ref[...] = f[:, 3 * _D:]
    vp = f @ We1_ref[...] + ve_ref[...] @ We2_ref[...] + be_ref[...]
    vepre_ref[...] = vp

    @pl.when(pl.program_id(0) == 0)
    def _():
        stats_ref[...] = jnp.zeros_like(stats_ref)

    s = jnp.sum(vp, axis=0, keepdims=True)
    s2 = jnp.sum(vp * vp, axis=0, keepdims=True)
    stats_ref[0:1, :] += s
    stats_ref[1:2, :] += s2


def _node_kernel(vc_ref, nf1_ref, nf2_ref, nf3_ref, nf4_ref, Wr_ref, br_ref,
                 out_ref, stats_ref):
    Wr = Wr_ref[...]
    o = (vc_ref[...] @ Wr[0:_D] + nf1_ref[...] @ Wr[_D:2 * _D]
         + nf2_ref[...] @ Wr[2 * _D:3 * _D] + nf3_ref[...] @ Wr[3 * _D:4 * _D]
         + nf4_ref[...] @ Wr[4 * _D:5 * _D] + br_ref[...])
    out_ref[...] = o

    @pl.when(pl.program_id(0) == 0)
    def _():
        stats_ref[...] = jnp.zeros_like(stats_ref)

    stats_ref[0:1, :] += jnp.sum(o, axis=0, keepdims=True)
    stats_ref[1:2, :] += jnp.sum(o * o, axis=0, keepdims=True)


def _norm_kernel(x_ref, scale_ref, shift_ref, o_ref):
    o_ref[...] = x_ref[...] * scale_ref[...] + shift_ref[...]


def _normalize(x, scale, shift, blk):
    n = x.shape[0]
    return pl.pallas_call(
        _norm_kernel,
        grid=(n // blk,),
        in_specs=[
            pl.BlockSpec((blk, _D), lambda i: (i, 0)),
            pl.BlockSpec((1, _D), lambda i: (0, 0)),
            pl.BlockSpec((1, _D), lambda i: (0, 0)),
        ],
        out_specs=pl.BlockSpec((blk, _D), lambda i: (i, 0)),
        out_shape=jax.ShapeDtypeStruct((n, _D), jnp.float32),
    )(x, scale, shift)


def kernel(in_vc, in_ve, edge_index, W1, b1, W2, b2, Wr, br, We, be,
           gamma_gc, beta_gc, gamma_ef, beta_ef):
    src = edge_index[0]
    dst = edge_index[1]

    W2p = jnp.concatenate(
        [W2[:, 1:], W2[:, 0:1], jnp.zeros((3 * _D, _D - 1), jnp.float32)], axis=1)
    b2p = jnp.concatenate(
        [b2[1:], b2[0:1], jnp.zeros((_D - 1,), jnp.float32)])[None, :]
    b1r = b1[None, :]
    ber = be[None, :]
    brr = br[None, :]
    We1 = We[:4 * _D]
    We2 = We[4 * _D:]

    sdst, perm = lax.sort_key_val(dst, lax.iota(jnp.int32, _E))

    a, b, veg = _sc_gather_sorted(in_vc, in_ve, src, sdst, perm)

    deg = jax.ops.segment_sum(jnp.ones((_E,), jnp.float32), dst,
                              num_segments=_N)

    half = _E // 2
    nhb = half // _EBLK

    def _edge_half(off):
        return pl.pallas_call(
            _edge_kernel,
            grid=(nhb,),
            in_specs=[
                pl.BlockSpec((_EBLK, _D), lambda i: (i + off, 0)),
                pl.BlockSpec((_EBLK, _D), lambda i: (i + off, 0)),
                pl.BlockSpec((_EBLK, _D), lambda i: (i + off, 0)),
                pl.BlockSpec((3 * _D, 3 * _D), lambda i: (0, 0)),
                pl.BlockSpec((1, 3 * _D), lambda i: (0, 0)),
                pl.BlockSpec((3 * _D, 5 * _D), lambda i: (0, 0)),
                pl.BlockSpec((1, 5 * _D), lambda i: (0, 0)),
                pl.BlockSpec((4 * _D, _D), lambda i: (0, 0)),
                pl.BlockSpec((_D, _D), lambda i: (0, 0)),
                pl.BlockSpec((1, _D), lambda i: (0, 0)),
            ],
            out_specs=[
                pl.BlockSpec((_EBLK, _D), lambda i: (i, 0)),
                pl.BlockSpec((_EBLK, _D), lambda i: (i, 0)),
                pl.BlockSpec((_EBLK, _D), lambda i: (i, 0)),
                pl.BlockSpec((_EBLK, _D), lambda i: (i, 0)),
                pl.BlockSpec((_EBLK, _D), lambda i: (i, 0)),
                pl.BlockSpec((8, _D), lambda i: (0, 0)),
            ],
            out_shape=[
                jax.ShapeDtypeStruct((half, _D), jnp.float32),
                jax.ShapeDtypeStruct((half, _D), jnp.float32),
                jax.ShapeDtypeStruct((half, _D), jnp.float32),
                jax.ShapeDtypeStruct((half, _D), jnp.float32),
                jax.ShapeDtypeStruct((half, _D), jnp.float32),
                jax.ShapeDtypeStruct((8, _D), jnp.float32),
            ],
        )(a, b, veg, W1, b1r, W2p, b2p, We1, We2, ber)

    sdst1 = sdst[:half]
    sdst2 = sdst[half:]
    f1a, f2a, f3a, f4a, vpa, esta = _edge_half(0)
    C1a, C4a = _cumsum2(f1a, f4a)
    m2a = jax.ops.segment_max(f2a, sdst1, num_segments=_N,
                              indices_are_sorted=True)
    m3a = jax.ops.segment_min(f3a, sdst1, num_segments=_N,
                              indices_are_sorted=True)
    f1b, f2b, f3b, f4b, vpb, estb = _edge_half(nhb)
    C1b, C4b = _cumsum2(f1b, f4b)
    m2b = jax.ops.segment_max(f2b, sdst2, num_segments=_N,
                              indices_are_sorted=True)
    m3b = jax.ops.segment_min(f3b, sdst2, num_segments=_N,
                              indices_are_sorted=True)

    rp_end = jnp.cumsum(deg).astype(jnp.int32)
    rp_start = rp_end - deg.astype(jnp.int32)

    T1a = C1a[half - 1]
    T4a = C4a[half - 1]

    rp_full = jnp.concatenate(
        [jnp.zeros((1,), jnp.int32), rp_end,
         jnp.zeros((_NPAD - _N - 1,), jnp.int32)])
    ia = jnp.clip(rp_full - 1, 0, half - 1)
    ib = jnp.clip(rp_full - half - 1, 0, half - 1)
    v1a, v1b, v4a, v4b = _sc_boundary_gather(C1a, C1b, C4a, C4b, ia, ib)

    def _cum_at(va, vb, Ta):
        v = jnp.where((rp_full > half)[:, None], vb + Ta[None, :], va)
        return jnp.where((rp_full == 0)[:, None], 0.0, v)

    g1 = _cum_at(v1a, v1b, T1a)
    g4 = _cum_at(v4a, v4b, T4a)
    nf1 = g1[1:_N + 1] - g1[:_N]
    s4 = g4[1:_N + 1] - g4[:_N]

    vepre = jnp.concatenate([vpa, vpb], axis=0)
    est = esta + estb

    has = (deg > 0)[:, None]
    nf2 = jnp.where(has, jnp.maximum(m2a, m2b), 0.0)
    nf3 = jnp.where(has, jnp.minimum(m3a, m3b), 0.0)
    nf4 = s4 / jnp.maximum(deg, 1.0)[:, None]

    vcpre, nst = pl.pallas_call(
        _node_kernel,
        grid=(_N // _NBLK,),
        in_specs=[
            pl.BlockSpec((_NBLK, _D), lambda i: (i, 0)),
            pl.BlockSpec((_NBLK, _D), lambda i: (i, 0)),
            pl.BlockSpec((_NBLK, _D), lambda i: (i, 0)),
            pl.BlockSpec((_NBLK, _D), lambda i: (i, 0)),
            pl.BlockSpec((_NBLK, _D), lambda i: (i, 0)),
            pl.BlockSpec((5 * _D, _D), lambda i: (0, 0)),
            pl.BlockSpec((1, _D), lambda i: (0, 0)),
        ],
        out_specs=[
            pl.BlockSpec((_NBLK, _D), lambda i: (i, 0)),
            pl.BlockSpec((8, _D), lambda i: (0, 0)),
        ],
        out_shape=[
            jax.ShapeDtypeStruct((_N, _D), jnp.float32),
            jax.ShapeDtypeStruct((8, _D), jnp.float32),
        ],
    )(in_vc, nf1, nf2, nf3, nf4, Wr, brr)

    def _bn_coeffs(stats, n, gamma, beta):
        mu = stats[0] / n
        var = stats[1] / n - mu * mu
        rstd = jax.lax.rsqrt(var + _EPS)
        scale = gamma * rstd
        shift = beta - mu * scale
        return scale[None, :], shift[None, :]

    esc, esh = _bn_coeffs(est, float(_E), gamma_ef, beta_ef)
    nsc, nsh = _bn_coeffs(nst, float(_N), gamma_gc, beta_gc)

    out_vc = _normalize(vcpre, nsc, nsh, _NBLK)
    out_ve = _sc_unpermute(_normalize(vepre, esc, esh, 2000), perm)
    return (out_vc, out_ve)


# speedup vs baseline: 1.7590x; 1.0373x over previous
import functools
import jax
import jax.numpy as jnp
from jax import lax
from jax.experimental import pallas as pl
from jax.experimental.pallas import tpu as pltpu
from jax.experimental.pallas import tpu_sc as plsc

_D = 128
_N = 10000
_E = 320000
_EPS = 1e-5
_EBLK = 640
_NBLK = 1000

_NW = 32
_GCHUNK = 400


def _sc_gather_sorted(table, ve, src, sdst, perm):
    n = sdst.shape[0]
    per_w = n // _NW
    chunk = 200
    n_it = per_w // chunk
    mesh = plsc.VectorSubcoreMesh(core_axis_name="c", subcore_axis_name="s")

    @functools.partial(
        pl.kernel,
        out_type=[jax.ShapeDtypeStruct((n, _D), jnp.float32),
                  jax.ShapeDtypeStruct((n, _D), jnp.float32),
                  jax.ShapeDtypeStruct((n, _D), jnp.float32)],
        mesh=mesh,
        scratch_types=[
            pltpu.VMEM((chunk,), jnp.int32),
            pltpu.VMEM((chunk,), jnp.int32),
            pltpu.VMEM((chunk,), jnp.int32),
            pltpu.VMEM((chunk, _D), jnp.float32),
            pltpu.VMEM((chunk, _D), jnp.float32),
            pltpu.VMEM((chunk, _D), jnp.float32),
            pltpu.SemaphoreType.DMA,
            pltpu.SemaphoreType.DMA,
            pltpu.SemaphoreType.DMA,
        ],
    )
    def k(table_hbm, ve_hbm, src_hbm, sdst_hbm, perm_hbm,
          a_hbm, b_hbm, veg_hbm, perm_v, sdst_v, srcg_v,
          ra_v, rb_v, rv_v, sa, sb, sv):
        wid = lax.axis_index("s") * 2 + lax.axis_index("c")
        base0 = wid * per_w

        def body(j, carry):
            base = base0 + j * chunk
            sl = pl.ds(base, chunk)
            pltpu.sync_copy(perm_hbm.at[sl], perm_v)
            pltpu.sync_copy(sdst_hbm.at[sl], sdst_v)
            gb = pltpu.async_copy(table_hbm.at[sdst_v], rb_v, sb)
            gv = pltpu.async_copy(ve_hbm.at[perm_v], rv_v, sv)
            pltpu.async_copy(src_hbm.at[perm_v], srcg_v, sa).wait()
            ga = pltpu.async_copy(table_hbm.at[srcg_v], ra_v, sa)
            gb.wait()
            wb = pltpu.async_copy(rb_v, b_hbm.at[sl], sb)
            gv.wait()
            wv = pltpu.async_copy(rv_v, veg_hbm.at[sl], sv)
            ga.wait()
            wa = pltpu.async_copy(ra_v, a_hbm.at[sl], sa)
            wb.wait()
            wv.wait()
            wa.wait()
            return carry

        lax.fori_loop(0, n_it, body, 0)

    return k(table, ve, src, sdst, perm)


def _sc_unpermute(x, perm):
    per_w = _E // _NW
    n_it = per_w // _GCHUNK
    mesh = plsc.VectorSubcoreMesh(core_axis_name="c", subcore_axis_name="s")

    @functools.partial(
        pl.kernel,
        out_type=jax.ShapeDtypeStruct((_E, _D), jnp.float32),
        mesh=mesh,
        scratch_types=[
            pltpu.VMEM((_GCHUNK,), jnp.int32),
            pltpu.VMEM((_GCHUNK, _D), jnp.float32),
            pltpu.SemaphoreType.DMA,
        ],
    )
    def k(x_hbm, perm_hbm, out_hbm, perm_v, rows_v, sem):
        wid = lax.axis_index("s") * 2 + lax.axis_index("c")
        base0 = wid * per_w

        def body(j, carry):
            base = base0 + j * _GCHUNK
            sl = pl.ds(base, _GCHUNK)
            pltpu.sync_copy(perm_hbm.at[sl], perm_v)
            pltpu.sync_copy(x_hbm.at[sl], rows_v)
            pltpu.async_copy(rows_v, out_hbm.at[perm_v], sem).wait()
            return carry

        lax.fori_loop(0, n_it, body, 0)

    return k(x, perm)


def _inclusive_cumsum(x):
    n = x.shape[0]
    s = x
    shift = 1
    while shift < n:
        z = jnp.zeros((shift, x.shape[1]), x.dtype)
        s = s + jnp.concatenate([z, s[:-shift]], axis=0)
        shift *= 2
    return s


_NPAD = 10240


def _sc_boundary_gather(C1a, C1b, C4a, C4b, ia, ib):
    per_w = _NPAD // _NW
    chunk = 160
    mesh = plsc.VectorSubcoreMesh(core_axis_name="c", subcore_axis_name="s")
    ot = jax.ShapeDtypeStruct((_NPAD, _D), jnp.float32)

    @functools.partial(
        pl.kernel,
        out_type=[ot] * 4,
        mesh=mesh,
        scratch_types=[
            pltpu.VMEM((chunk,), jnp.int32),
            pltpu.VMEM((chunk,), jnp.int32),
            pltpu.VMEM((chunk, _D), jnp.float32),
            pltpu.VMEM((chunk, _D), jnp.float32),
            pltpu.VMEM((chunk, _D), jnp.float32),
            pltpu.VMEM((chunk, _D), jnp.float32),
            pltpu.SemaphoreType.DMA,
            pltpu.SemaphoreType.DMA,
            pltpu.SemaphoreType.DMA,
            pltpu.SemaphoreType.DMA,
        ],
    )
    def k(c1a, c1b, c4a, c4b, iah, ibh,
          o1a, o1b, o4a, o4b, ia_v, ib_v, r1a, r1b, r4a, r4b,
          s1, s2, s3, s4):
        wid = lax.axis_index("s") * 2 + lax.axis_index("c")
        base0 = wid * per_w

        def body(j, carry):
            sl = pl.ds(base0 + j * chunk, chunk)
            pltpu.sync_copy(iah.at[sl], ia_v)
            pltpu.sync_copy(ibh.at[sl], ib_v)
            g1 = pltpu.async_copy(c1a.at[ia_v], r1a, s1)
            g2 = pltpu.async_copy(c1b.at[ib_v], r1b, s2)
            g3 = pltpu.async_copy(c4a.at[ia_v], r4a, s3)
            g4 = pltpu.async_copy(c4b.at[ib_v], r4b, s4)
            g1.wait()
            w1 = pltpu.async_copy(r1a, o1a.at[sl], s1)
            g2.wait()
            w2 = pltpu.async_copy(r1b, o1b.at[sl], s2)
            g3.wait()
            w3 = pltpu.async_copy(r4a, o4a.at[sl], s3)
            g4.wait()
            w4 = pltpu.async_copy(r4b, o4b.at[sl], s4)
            w1.wait()
            w2.wait()
            w3.wait()
            w4.wait()
            return carry

        lax.fori_loop(0, per_w // chunk, body, 0)

    return k(C1a, C1b, C4a, C4b, ia, ib)


def _cumsum_kernel(x1_ref, x4_ref, c1_ref, c4_ref, carry_ref):
    @pl.when(pl.program_id(0) == 0)
    def _():
        carry_ref[...] = jnp.zeros_like(carry_ref)

    cs1 = _inclusive_cumsum(x1_ref[...]) + carry_ref[0:1, :]
    c1_ref[...] = cs1
    carry_ref[0:1, :] = cs1[-1:, :]
    cs4 = _inclusive_cumsum(x4_ref[...]) + carry_ref[1:2, :]
    c4_ref[...] = cs4
    carry_ref[1:2, :] = cs4[-1:, :]


def _cumsum2(x1, x4):
    n = x1.shape[0]
    return pl.pallas_call(
        _cumsum_kernel,
        grid=(n // _EBLK,),
        in_specs=[
            pl.BlockSpec((_EBLK, _D), lambda i: (i, 0)),
            pl.BlockSpec((_EBLK, _D), lambda i: (i, 0)),
        ],
        out_specs=[
            pl.BlockSpec((_EBLK, _D), lambda i: (i, 0)),
            pl.BlockSpec((_EBLK, _D), lambda i: (i, 0)),
        ],
        out_shape=[
            jax.ShapeDtypeStruct((n, _D), jnp.float32),
            jax.ShapeDtypeStruct((n, _D), jnp.float32),
        ],
        scratch_shapes=[pltpu.VMEM((8, _D), jnp.float32)],
    )(x1, x4)


def _edge_kernel(a_ref, b_ref, ve_ref, W1_ref, b1_ref, W2p_ref, b2p_ref,
                 We1_ref, We2_ref, be_ref, f1_ref, f2_ref, f3_ref, f4_ref,
                 vepre_ref, stats_ref):
    x = jnp.concatenate([a_ref[...], b_ref[...], ve_ref[...]], axis=1)
    h = jnp.maximum(x @ W1_ref[...] + b1_ref[...], 0.0)
    m = h @ W2p_ref[...] + b2p_ref[...]
    k = jax.nn.sigmoid(m[:, 4 * _D:4 * _D + 1])
    f = m[:, :4 * _D] * k
    f1_ref[...] = f[:, :_D]
    f2_ref[...] = f[:, _D:2 * _D]
    f3_ref[...] = f[:, 2 * _D:3 * _D]
    f4_ref[...] = f[:, 3 * _D:]
    vp = f @ We1_ref[...] + ve_ref[...] @ We2_ref[...] + be_ref[...]
    vepre_ref[...] = vp

    @pl.when(pl.program_id(0) == 0)
    def _():
        stats_ref[...] = jnp.zeros_like(stats_ref)

    s = jnp.sum(vp, axis=0, keepdims=True)
    s2 = jnp.sum(vp * vp, axis=0, keepdims=True)
    stats_ref[0:1, :] += s
    stats_ref[1:2, :] += s2


def _node_kernel(vc_ref, nf1_ref, nf2_ref, nf3_ref, nf4_ref, Wr_ref, br_ref,
                 out_ref, stats_ref):
    Wr = Wr_ref[...]
    o = (vc_ref[...] @ Wr[0:_D] + nf1_ref[...] @ Wr[_D:2 * _D]
         + nf2_ref[...] @ Wr[2 * _D:3 * _D] + nf3_ref[...] @ Wr[3 * _D:4 * _D]
         + nf4_ref[...] @ Wr[4 * _D:5 * _D] + br_ref[...])
    out_ref[...] = o

    @pl.when(pl.program_id(0) == 0)
    def _():
        stats_ref[...] = jnp.zeros_like(stats_ref)

    stats_ref[0:1, :] += jnp.sum(o, axis=0, keepdims=True)
    stats_ref[1:2, :] += jnp.sum(o * o, axis=0, keepdims=True)


def _norm_kernel(x_ref, scale_ref, shift_ref, o_ref):
    o_ref[...] = x_ref[...] * scale_ref[...] + shift_ref[...]


def _normalize(x, scale, shift, blk):
    n = x.shape[0]
    return pl.pallas_call(
        _norm_kernel,
        grid=(n // blk,),
        in_specs=[
            pl.BlockSpec((blk, _D), lambda i: (i, 0)),
            pl.BlockSpec((1, _D), lambda i: (0, 0)),
            pl.BlockSpec((1, _D), lambda i: (0, 0)),
        ],
        out_specs=pl.BlockSpec((blk, _D), lambda i: (i, 0)),
        out_shape=jax.ShapeDtypeStruct((n, _D), jnp.float32),
    )(x, scale, shift)


def kernel(in_vc, in_ve, edge_index, W1, b1, W2, b2, Wr, br, We, be,
           gamma_gc, beta_gc, gamma_ef, beta_ef):
    src = edge_index[0]
    dst = edge_index[1]

    W2p = jnp.concatenate(
        [W2[:, 1:], W2[:, 0:1], jnp.zeros((3 * _D, _D - 1), jnp.float32)], axis=1)
    b2p = jnp.concatenate(
        [b2[1:], b2[0:1], jnp.zeros((_D - 1,), jnp.float32)])[None, :]
    b1r = b1[None, :]
    ber = be[None, :]
    brr = br[None, :]
    We1 = We[:4 * _D]
    We2 = We[4 * _D:]

    sdst, perm = lax.sort_key_val(dst, lax.iota(jnp.int32, _E))

    deg = jax.ops.segment_sum(jnp.ones((_E,), jnp.float32), dst,
                              num_segments=_N)

    half = _E // 2
    nhb = half // _EBLK

    def _edge_half(ah, bh, vh):
        return pl.pallas_call(
            _edge_kernel,
            grid=(nhb,),
            in_specs=[
                pl.BlockSpec((_EBLK, _D), lambda i: (i, 0)),
                pl.BlockSpec((_EBLK, _D), lambda i: (i, 0)),
                pl.BlockSpec((_EBLK, _D), lambda i: (i, 0)),
                pl.BlockSpec((3 * _D, 3 * _D), lambda i: (0, 0)),
                pl.BlockSpec((1, 3 * _D), lambda i: (0, 0)),
                pl.BlockSpec((3 * _D, 5 * _D), lambda i: (0, 0)),
                pl.BlockSpec((1, 5 * _D), lambda i: (0, 0)),
                pl.BlockSpec((4 * _D, _D), lambda i: (0, 0)),
                pl.BlockSpec((_D, _D), lambda i: (0, 0)),
                pl.BlockSpec((1, _D), lambda i: (0, 0)),
            ],
            out_specs=[
                pl.BlockSpec((_EBLK, _D), lambda i: (i, 0)),
                pl.BlockSpec((_EBLK, _D), lambda i: (i, 0)),
                pl.BlockSpec((_EBLK, _D), lambda i: (i, 0)),
                pl.BlockSpec((_EBLK, _D), lambda i: (i, 0)),
                pl.BlockSpec((_EBLK, _D), lambda i: (i, 0)),
                pl.BlockSpec((8, _D), lambda i: (0, 0)),
            ],
            out_shape=[
                jax.ShapeDtypeStruct((half, _D), jnp.float32),
                jax.ShapeDtypeStruct((half, _D), jnp.float32),
                jax.ShapeDtypeStruct((half, _D), jnp.float32),
                jax.ShapeDtypeStruct((half, _D), jnp.float32),
                jax.ShapeDtypeStruct((half, _D), jnp.float32),
                jax.ShapeDtypeStruct((8, _D), jnp.float32),
            ],
        )(ah, bh, vh, W1, b1r, W2p, b2p, We1, We2, ber)

    sdst1 = sdst[:half]
    sdst2 = sdst[half:]
    ga1 = _sc_gather_sorted(in_vc, in_ve, src, sdst1, perm[:half])
    f1a, f2a, f3a, f4a, vpa, esta = _edge_half(*ga1)
    ga2 = _sc_gather_sorted(in_vc, in_ve, src, sdst2, perm[half:])
    C1a, C4a = _cumsum2(f1a, f4a)
    m2a = jax.ops.segment_max(f2a, sdst1, num_segments=_N,
                              indices_are_sorted=True)
    m3a = jax.ops.segment_min(f3a, sdst1, num_segments=_N,
                              indices_are_sorted=True)
    f1b, f2b, f3b, f4b, vpb, estb = _edge_half(*ga2)
    C1b, C4b = _cumsum2(f1b, f4b)
    m2b = jax.ops.segment_max(f2b, sdst2, num_segments=_N,
                              indices_are_sorted=True)
    m3b = jax.ops.segment_min(f3b, sdst2, num_segments=_N,
                              indices_are_sorted=True)

    rp_end = jnp.cumsum(deg).astype(jnp.int32)
    rp_start = rp_end - deg.astype(jnp.int32)

    T1a = C1a[half - 1]
    T4a = C4a[half - 1]

    rp_full = jnp.concatenate(
        [jnp.zeros((1,), jnp.int32), rp_end,
         jnp.zeros((_NPAD - _N - 1,), jnp.int32)])
    ia = jnp.clip(rp_full - 1, 0, half - 1)
    ib = jnp.clip(rp_full - half - 1, 0, half - 1)
    v1a, v1b, v4a, v4b = _sc_boundary_gather(C1a, C1b, C4a, C4b, ia, ib)

    def _cum_at(va, vb, Ta):
        v = jnp.where((rp_full > half)[:, None], vb + Ta[None, :], va)
        return jnp.where((rp_full == 0)[:, None], 0.0, v)

    g1 = _cum_at(v1a, v1b, T1a)
    g4 = _cum_at(v4a, v4b, T4a)
    nf1 = g1[1:_N + 1] - g1[:_N]
    s4 = g4[1:_N + 1] - g4[:_N]

    vepre = jnp.concatenate([vpa, vpb], axis=0)
    est = esta + estb

    has = (deg > 0)[:, None]
    nf2 = jnp.where(has, jnp.maximum(m2a, m2b), 0.0)
    nf3 = jnp.where(has, jnp.minimum(m3a, m3b), 0.0)
    nf4 = s4 / jnp.maximum(deg, 1.0)[:, None]

    vcpre, nst = pl.pallas_call(
        _node_kernel,
        grid=(_N // _NBLK,),
        in_specs=[
            pl.BlockSpec((_NBLK, _D), lambda i: (i, 0)),
            pl.BlockSpec((_NBLK, _D), lambda i: (i, 0)),
            pl.BlockSpec((_NBLK, _D), lambda i: (i, 0)),
            pl.BlockSpec((_NBLK, _D), lambda i: (i, 0)),
            pl.BlockSpec((_NBLK, _D), lambda i: (i, 0)),
            pl.BlockSpec((5 * _D, _D), lambda i: (0, 0)),
            pl.BlockSpec((1, _D), lambda i: (0, 0)),
        ],
        out_specs=[
            pl.BlockSpec((_NBLK, _D), lambda i: (i, 0)),
            pl.BlockSpec((8, _D), lambda i: (0, 0)),
        ],
        out_shape=[
            jax.ShapeDtypeStruct((_N, _D), jnp.float32),
            jax.ShapeDtypeStruct((8, _D), jnp.float32),
        ],
    )(in_vc, nf1, nf2, nf3, nf4, Wr, brr)

    def _bn_coeffs(stats, n, gamma, beta):
        mu = stats[0] / n
        var = stats[1] / n - mu * mu
        rstd = jax.lax.rsqrt(var + _EPS)
        scale = gamma * rstd
        shift = beta - mu * scale
        return scale[None, :], shift[None, :]

    esc, esh = _bn_coeffs(est, float(_E), gamma_ef, beta_ef)
    nsc, nsh = _bn_coeffs(nst, float(_N), gamma_gc, beta_gc)

    out_vc = _normalize(vcpre, nsc, nsh, _NBLK)
    out_ve = _sc_unpermute(_normalize(vepre, esc, esh, 2000), perm)
    return (out_vc, out_ve)
